# baseline (speedup 1.0000x reference)
import jax
import jax.numpy as jnp
from jax.experimental import pallas as pl
from jax.experimental.pallas import tpu as pltpu

_Q_THRESH = float.fromhex("0x1.47ae16p-1")
_B = 1024
_K = 16384
_BB = 32
_CW = 128


def _prep_kernel(craw_ref, ct_ref, cdirn_ref, rhs5_ref):
    craw = craw_ref[...]
    cn = jnp.sqrt(jnp.sum(craw * craw, axis=0, keepdims=True))
    cdirn_ref[...] = craw / (cn + 1e-8)
    ct = ct_ref[...]
    rhs5_ref[0:3, :] = ct
    rhs5_ref[3:4, :] = jnp.ones_like(ct[0:1, :])
    rhs5_ref[4:5, :] = jnp.sum(ct * ct, axis=0, keepdims=True)


def _main_kernel(cdirn_ref, rhs5_ref, tpf_ref, out_ref, idx_ref, cnt_ref,
                 cos_scr, q_scr):
    tpf = tpf_ref[...]

    tdr = jnp.concatenate([tpf[:, 2:3], tpf[:, 6:7], tpf[:, 10:11]], axis=1)
    tn = jnp.sqrt(jnp.sum(tdr * tdr, axis=1, keepdims=True))
    tdir = tdr / (tn + 1e-8)
    torig = jnp.concatenate([tpf[:, 3:4], tpf[:, 7:8], tpf[:, 11:12]], axis=1)
    o2 = jnp.sum(torig * torig, axis=1, keepdims=True)

    cos_scr[...] = jax.lax.dot_general(
        tdir, cdirn_ref[...], (((1,), (0,)), ((), ())),
        preferred_element_type=jnp.float32)
    lhs5 = jnp.concatenate(
        [-2.0 * torig, o2, jnp.ones_like(o2)], axis=1)
    q_scr[...] = jax.lax.dot_general(
        lhs5, rhs5_ref[...], (((1,), (0,)), ((), ())),
        preferred_element_type=jnp.float32)

    lane = jax.lax.broadcasted_iota(jnp.int32, (_BB, _CW), 1)
    neg4 = jnp.full((_BB, _CW), -4.0, jnp.float32)
    sent = jnp.full((_BB, _CW), _K, jnp.int32)
    zf = jnp.zeros((_BB, _CW), jnp.float32)
    zi = jnp.zeros((_BB, _CW), jnp.int32)

    def chunk(i, st):
        m_i, i_i, q_i, m_a, i_a, q_a, cnt = st
        sl = pl.ds(i * _CW, _CW)
        cos = jnp.clip(cos_scr[:, sl], -0.999999, 0.999999)
        q = q_scr[:, sl]
        inr = q <= _Q_THRESH
        cnt = cnt + jnp.where(inr, 1, 0)
        idxv = lane + i * _CW
        eff = jnp.where(inr, cos, -3.0)
        u = eff > m_i
        m_i = jnp.where(u, eff, m_i)
        i_i = jnp.where(u, idxv, i_i)
        q_i = jnp.where(u, q, q_i)
        v = cos > m_a
        m_a = jnp.where(v, cos, m_a)
        i_a = jnp.where(v, idxv, i_a)
        q_a = jnp.where(v, q, q_a)
        return (m_i, i_i, q_i, m_a, i_a, q_a, cnt)

    m_i, i_i, q_i, m_a, i_a, q_a, cntl = jax.lax.fori_loop(
        0, _K // _CW, chunk, (neg4, sent, zf, neg4, sent, zf, zi))

    cnt = jnp.sum(cntl, axis=1, keepdims=True)
    any_in = cnt > 0
    m_l = jnp.where(any_in, m_i, m_a)
    i_l = jnp.where(any_in, i_i, i_a)
    q_l = jnp.where(any_in, q_i, q_a)
    m = jnp.max(m_l, axis=1, keepdims=True)
    idx = jnp.min(jnp.where(m_l == m, i_l, _K), axis=1, keepdims=True)
    qsel = jnp.max(jnp.where(i_l == idx, q_l, -1.0), axis=1, keepdims=True)

    angle = 2.0 * jnp.arctan2(jnp.sqrt(1.0 - m * m), 1.0 + m)
    dist = jnp.sqrt(jnp.maximum(qsel, 0.0) + 1e-12)

    tr9 = jnp.concatenate([tpf[:, 0:3], tpf[:, 4:7], tpf[:, 8:11]], axis=1)
    out_ref[...] = jnp.concatenate([angle, dist, tr9], axis=1)
    idx_ref[...] = idx
    cnt_ref[...] = cnt


def kernel(candidate_rotations, candidate_translations, target_pose):
    craw3 = candidate_rotations[:, :, 2].T
    ct3 = candidate_translations.T
    tpf = target_pose.reshape(_B, 16)

    cdirn, rhs5 = pl.pallas_call(
        _prep_kernel,
        out_shape=[
            jax.ShapeDtypeStruct((3, _K), jnp.float32),
            jax.ShapeDtypeStruct((5, _K), jnp.float32),
        ],
    )(craw3, ct3)

    out_f, idx, cnt = pl.pallas_call(
        _main_kernel,
        grid=(_B // _BB,),
        in_specs=[
            pl.BlockSpec((3, _K), lambda b: (0, 0)),
            pl.BlockSpec((5, _K), lambda b: (0, 0)),
            pl.BlockSpec((_BB, 16), lambda b: (b, 0)),
        ],
        out_specs=[
            pl.BlockSpec((_BB, 11), lambda b: (b, 0)),
            pl.BlockSpec((_BB, 1), lambda b: (b, 0)),
            pl.BlockSpec((_BB, 1), lambda b: (b, 0)),
        ],
        out_shape=[
            jax.ShapeDtypeStruct((_B, 11), jnp.float32),
            jax.ShapeDtypeStruct((_B, 1), jnp.int32),
            jax.ShapeDtypeStruct((_B, 1), jnp.int32),
        ],
        scratch_shapes=[
            pltpu.VMEM((_BB, _K), jnp.float32),
            pltpu.VMEM((_BB, _K), jnp.float32),
        ],
        compiler_params=pltpu.CompilerParams(
            dimension_semantics=("parallel",),
        ),
    )(cdirn, rhs5, tpf)
    return out_f, idx[:, 0], cnt[:, 0]


# speedup vs baseline: 1.3452x; 1.3452x over previous
import jax
import jax.numpy as jnp
from jax.experimental import pallas as pl
from jax.experimental.pallas import tpu as pltpu

_Q_THRESH = float.fromhex("0x1.47ae16p-1")
_B = 1024
_K = 16384
_BB = 32
_CW = 128


def _prep_kernel(craw_ref, ct_ref, cdirn_ref, rhs5_ref):
    craw = craw_ref[...]
    cn = jnp.sqrt(jnp.sum(craw * craw, axis=0, keepdims=True))
    cdirn_ref[...] = craw / (cn + 1e-8)
    ct = ct_ref[...]
    rhs5_ref[0:3, :] = ct
    rhs5_ref[3:4, :] = jnp.sum(ct * ct, axis=0, keepdims=True)


def _main_kernel(cdirn_ref, rhs4_ref, tpf_ref, out_ref, idx_ref, cnt_ref,
                 cos_scr, q_scr):
    tpf = tpf_ref[...]

    tdr = jnp.concatenate([tpf[:, 2:3], tpf[:, 6:7], tpf[:, 10:11]], axis=1)
    tn = jnp.sqrt(jnp.sum(tdr * tdr, axis=1, keepdims=True))
    tdir = tdr / (tn + 1e-8)
    torig = jnp.concatenate([tpf[:, 3:4], tpf[:, 7:8], tpf[:, 11:12]], axis=1)
    o2 = jnp.sum(torig * torig, axis=1, keepdims=True)

    cos_scr[...] = jax.lax.dot_general(
        tdir, cdirn_ref[...], (((1,), (0,)), ((), ())),
        preferred_element_type=jnp.float32)
    q_scr[...] = jax.lax.dot_general(
        torig, rhs4_ref[0:3, :], (((1,), (0,)), ((), ())),
        preferred_element_type=jnp.float32)

    lane = jax.lax.broadcasted_iota(jnp.int32, (_BB, _CW), 1)
    neg4 = jnp.full((_BB, _CW), -4.0, jnp.float32)
    sent = jnp.full((_BB, _CW), _K, jnp.int32)
    zf = jnp.zeros((_BB, _CW), jnp.float32)
    zi = jnp.zeros((_BB, _CW), jnp.int32)

    def chunk(i, st):
        m_i, i_i, q_i, m_a, i_a, q_a, cnt = st
        sl = pl.ds(i * _CW, _CW)
        cos = jnp.clip(cos_scr[:, sl], -0.999999, 0.999999)
        q = (o2 + rhs4_ref[3:4, sl]) - 2.0 * q_scr[:, sl]
        inr = q <= _Q_THRESH
        cnt = cnt + jnp.where(inr, 1, 0)
        idxv = lane + i * _CW
        eff = jnp.where(inr, cos, -3.0)
        u = eff > m_i
        m_i = jnp.where(u, eff, m_i)
        i_i = jnp.where(u, idxv, i_i)
        q_i = jnp.where(u, q, q_i)
        v = cos > m_a
        m_a = jnp.where(v, cos, m_a)
        i_a = jnp.where(v, idxv, i_a)
        q_a = jnp.where(v, q, q_a)
        return (m_i, i_i, q_i, m_a, i_a, q_a, cnt)

    m_i, i_i, q_i, m_a, i_a, q_a, cntl = jax.lax.fori_loop(
        0, _K // _CW, chunk, (neg4, sent, zf, neg4, sent, zf, zi), unroll=8)

    cnt = jnp.sum(cntl, axis=1, keepdims=True)
    any_in = cnt > 0
    m_l = jnp.where(any_in, m_i, m_a)
    i_l = jnp.where(any_in, i_i, i_a)
    q_l = jnp.where(any_in, q_i, q_a)
    m = jnp.max(m_l, axis=1, keepdims=True)
    idx = jnp.min(jnp.where(m_l == m, i_l, _K), axis=1, keepdims=True)
    qsel = jnp.max(jnp.where(i_l == idx, q_l, -1.0), axis=1, keepdims=True)

    angle = 2.0 * jnp.arctan2(jnp.sqrt(1.0 - m * m), 1.0 + m)
    dist = jnp.sqrt(jnp.maximum(qsel, 0.0) + 1e-12)

    tr9 = jnp.concatenate([tpf[:, 0:3], tpf[:, 4:7], tpf[:, 8:11]], axis=1)
    out_ref[...] = jnp.concatenate([angle, dist, tr9], axis=1)
    idx_ref[...] = idx
    cnt_ref[...] = cnt


def kernel(candidate_rotations, candidate_translations, target_pose):
    craw3 = candidate_rotations[:, :, 2].T
    ct3 = candidate_translations.T
    tpf = target_pose.reshape(_B, 16)

    cdirn, rhs4 = pl.pallas_call(
        _prep_kernel,
        out_shape=[
            jax.ShapeDtypeStruct((3, _K), jnp.float32),
            jax.ShapeDtypeStruct((4, _K), jnp.float32),
        ],
    )(craw3, ct3)

    out_f, idx, cnt = pl.pallas_call(
        _main_kernel,
        grid=(_B // _BB,),
        in_specs=[
            pl.BlockSpec((3, _K), lambda b: (0, 0)),
            pl.BlockSpec((4, _K), lambda b: (0, 0)),
            pl.BlockSpec((_BB, 16), lambda b: (b, 0)),
        ],
        out_specs=[
            pl.BlockSpec((_BB, 11), lambda b: (b, 0)),
            pl.BlockSpec((_BB, 1), lambda b: (b, 0)),
            pl.BlockSpec((_BB, 1), lambda b: (b, 0)),
        ],
        out_shape=[
            jax.ShapeDtypeStruct((_B, 11), jnp.float32),
            jax.ShapeDtypeStruct((_B, 1), jnp.int32),
            jax.ShapeDtypeStruct((_B, 1), jnp.int32),
        ],
        scratch_shapes=[
            pltpu.VMEM((_BB, _K), jnp.float32),
            pltpu.VMEM((_BB, _K), jnp.float32),
        ],
        compiler_params=pltpu.CompilerParams(
            dimension_semantics=("parallel",),
        ),
    )(cdirn, rhs4, tpf)
    return out_f, idx[:, 0], cnt[:, 0]


# speedup vs baseline: 1.4073x; 1.0461x over previous
import jax
import jax.numpy as jnp
from jax.experimental import pallas as pl
from jax.experimental.pallas import tpu as pltpu

_Q_THRESH = float.fromhex("0x1.47ae16p-1")
_B = 1024
_K = 16384
_BB = 64
_CW = 128


def _prep_kernel(craw_ref, ct_ref, cdirn_ref, rhs5_ref):
    craw = craw_ref[...]
    cn = jnp.sqrt(jnp.sum(craw * craw, axis=0, keepdims=True))
    cdirn_ref[...] = craw / (cn + 1e-8)
    ct = ct_ref[...]
    rhs5_ref[0:3, :] = ct
    rhs5_ref[3:4, :] = jnp.sum(ct * ct, axis=0, keepdims=True)


def _main_kernel(cdirn_ref, rhs4_ref, tpf_ref, out_ref, idx_ref, cnt_ref,
                 cos_scr, q_scr):
    tpf = tpf_ref[...]

    tdr = jnp.concatenate([tpf[:, 2:3], tpf[:, 6:7], tpf[:, 10:11]], axis=1)
    tn = jnp.sqrt(jnp.sum(tdr * tdr, axis=1, keepdims=True))
    tdir = tdr / (tn + 1e-8)
    torig = jnp.concatenate([tpf[:, 3:4], tpf[:, 7:8], tpf[:, 11:12]], axis=1)
    o2 = jnp.sum(torig * torig, axis=1, keepdims=True)

    cos_scr[...] = jax.lax.dot_general(
        tdir, cdirn_ref[...], (((1,), (0,)), ((), ())),
        preferred_element_type=jnp.float32)
    q_scr[...] = jax.lax.dot_general(
        torig, rhs4_ref[0:3, :], (((1,), (0,)), ((), ())),
        preferred_element_type=jnp.float32)

    lane = jax.lax.broadcasted_iota(jnp.int32, (_BB, _CW), 1)
    neg4 = jnp.full((_BB, _CW), -4.0, jnp.float32)
    sent = jnp.full((_BB, _CW), _K, jnp.int32)
    zf = jnp.zeros((_BB, _CW), jnp.float32)
    zi = jnp.zeros((_BB, _CW), jnp.int32)

    def chunk(i, st):
        m_i, i_i, q_i, m_a, i_a, q_a, cnt = st
        sl = pl.ds(i * _CW, _CW)
        cos = jnp.clip(cos_scr[:, sl], -0.999999, 0.999999)
        q = (o2 + rhs4_ref[3:4, sl]) - 2.0 * q_scr[:, sl]
        inr = q <= _Q_THRESH
        cnt = cnt + jnp.where(inr, 1, 0)
        idxv = lane + i * _CW
        eff = jnp.where(inr, cos, -3.0)
        u = eff > m_i
        m_i = jnp.where(u, eff, m_i)
        i_i = jnp.where(u, idxv, i_i)
        q_i = jnp.where(u, q, q_i)
        v = cos > m_a
        m_a = jnp.where(v, cos, m_a)
        i_a = jnp.where(v, idxv, i_a)
        q_a = jnp.where(v, q, q_a)
        return (m_i, i_i, q_i, m_a, i_a, q_a, cnt)

    m_i, i_i, q_i, m_a, i_a, q_a, cntl = jax.lax.fori_loop(
        0, _K // _CW, chunk, (neg4, sent, zf, neg4, sent, zf, zi), unroll=8)

    cnt = jnp.sum(cntl, axis=1, keepdims=True)
    any_in = cnt > 0
    m_l = jnp.where(any_in, m_i, m_a)
    i_l = jnp.where(any_in, i_i, i_a)
    q_l = jnp.where(any_in, q_i, q_a)
    m = jnp.max(m_l, axis=1, keepdims=True)
    idx = jnp.min(jnp.where(m_l == m, i_l, _K), axis=1, keepdims=True)
    qsel = jnp.max(jnp.where(i_l == idx, q_l, -1.0), axis=1, keepdims=True)

    angle = 2.0 * jnp.arctan2(jnp.sqrt(1.0 - m * m), 1.0 + m)
    dist = jnp.sqrt(jnp.maximum(qsel, 0.0) + 1e-12)

    tr9 = jnp.concatenate([tpf[:, 0:3], tpf[:, 4:7], tpf[:, 8:11]], axis=1)
    out_ref[...] = jnp.concatenate([angle, dist, tr9], axis=1)
    idx_ref[...] = idx
    cnt_ref[...] = cnt


def kernel(candidate_rotations, candidate_translations, target_pose):
    craw3 = candidate_rotations[:, :, 2].T
    ct3 = candidate_translations.T
    tpf = target_pose.reshape(_B, 16)

    cdirn, rhs4 = pl.pallas_call(
        _prep_kernel,
        out_shape=[
            jax.ShapeDtypeStruct((3, _K), jnp.float32),
            jax.ShapeDtypeStruct((4, _K), jnp.float32),
        ],
    )(craw3, ct3)

    out_f, idx, cnt = pl.pallas_call(
        _main_kernel,
        grid=(_B // _BB,),
        in_specs=[
            pl.BlockSpec((3, _K), lambda b: (0, 0)),
            pl.BlockSpec((4, _K), lambda b: (0, 0)),
            pl.BlockSpec((_BB, 16), lambda b: (b, 0)),
        ],
        out_specs=[
            pl.BlockSpec((_BB, 11), lambda b: (b, 0)),
            pl.BlockSpec((_BB, 1), lambda b: (b, 0)),
            pl.BlockSpec((_BB, 1), lambda b: (b, 0)),
        ],
        out_shape=[
            jax.ShapeDtypeStruct((_B, 11), jnp.float32),
            jax.ShapeDtypeStruct((_B, 1), jnp.int32),
            jax.ShapeDtypeStruct((_B, 1), jnp.int32),
        ],
        scratch_shapes=[
            pltpu.VMEM((_BB, _K), jnp.float32),
            pltpu.VMEM((_BB, _K), jnp.float32),
        ],
        compiler_params=pltpu.CompilerParams(
            dimension_semantics=("parallel",),
        ),
    )(cdirn, rhs4, tpf)
    return out_f, idx[:, 0], cnt[:, 0]


# speedup vs baseline: 1.5261x; 1.0845x over previous
import jax
import jax.numpy as jnp
from jax.experimental import pallas as pl
from jax.experimental.pallas import tpu as pltpu

_Q_THRESH = float.fromhex("0x1.47ae16p-1")
_B = 1024
_K = 16384
_BB = 64
_CW = 128


def _prep_kernel(craw_ref, ct_ref, cdirn_ref, rhs5_ref):
    craw = craw_ref[...]
    cn = jnp.sqrt(jnp.sum(craw * craw, axis=0, keepdims=True))
    cdirn_ref[...] = craw / (cn + 1e-8)
    ct = ct_ref[...]
    rhs5_ref[0:3, :] = ct
    rhs5_ref[3:4, :] = jnp.sum(ct * ct, axis=0, keepdims=True)


def _main_kernel(cdirn_ref, rhs4_ref, tpf_ref, out_ref, idx_ref, cnt_ref,
                 cos_scr, q_scr):
    tpf = tpf_ref[...]

    tdr = jnp.concatenate([tpf[:, 2:3], tpf[:, 6:7], tpf[:, 10:11]], axis=1)
    tn = jnp.sqrt(jnp.sum(tdr * tdr, axis=1, keepdims=True))
    tdir = tdr / (tn + 1e-8)
    torig = jnp.concatenate([tpf[:, 3:4], tpf[:, 7:8], tpf[:, 11:12]], axis=1)
    o2 = jnp.sum(torig * torig, axis=1, keepdims=True)

    cos_scr[...] = jax.lax.dot_general(
        tdir, cdirn_ref[...], (((1,), (0,)), ((), ())),
        preferred_element_type=jnp.float32)
    q_scr[...] = jax.lax.dot_general(
        torig, rhs4_ref[0:3, :], (((1,), (0,)), ((), ())),
        preferred_element_type=jnp.float32)

    lane = jax.lax.broadcasted_iota(jnp.int32, (_BB, _CW), 1)
    neg4 = jnp.full((_BB, _CW), -4.0, jnp.float32)
    sent = jnp.full((_BB, _CW), _K, jnp.int32)
    zf = jnp.zeros((_BB, _CW), jnp.float32)
    zi = jnp.zeros((_BB, _CW), jnp.int32)

    def chunk(i, st):
        m_i, i_i, q_i, m_a, cnt = st
        sl = pl.ds(i * _CW, _CW)
        cos = jnp.clip(cos_scr[:, sl], -0.999999, 0.999999)
        q = (o2 + rhs4_ref[3:4, sl]) - 2.0 * q_scr[:, sl]
        inr = q <= _Q_THRESH
        cnt = cnt + jnp.where(inr, 1, 0)
        idxv = lane + i * _CW
        eff = jnp.where(inr, cos, -3.0)
        u = eff > m_i
        m_i = jnp.where(u, eff, m_i)
        i_i = jnp.where(u, idxv, i_i)
        q_i = jnp.where(u, q, q_i)
        m_a = jnp.maximum(m_a, cos)
        return (m_i, i_i, q_i, m_a, cnt)

    m_i, i_i, q_i, m_a, cntl = jax.lax.fori_loop(
        0, _K // _CW, chunk, (neg4, sent, zf, neg4, zi), unroll=8)

    cnt = jnp.sum(cntl, axis=1, keepdims=True)
    any_in = cnt > 0
    m = jnp.max(m_i, axis=1, keepdims=True)
    idx = jnp.min(jnp.where(m_i == m, i_i, _K), axis=1, keepdims=True)
    qsel = jnp.max(jnp.where(i_i == idx, q_i, -1.0), axis=1, keepdims=True)

    angle = 2.0 * jnp.arctan2(jnp.sqrt(1.0 - m * m), 1.0 + m)
    dist = jnp.sqrt(jnp.maximum(qsel, 0.0) + 1e-12)

    tr9 = jnp.concatenate([tpf[:, 0:3], tpf[:, 4:7], tpf[:, 8:11]], axis=1)
    out_ref[...] = jnp.concatenate([angle, dist, tr9], axis=1)
    idx_ref[...] = idx
    cnt_ref[...] = cnt

    @pl.when(jnp.min(cnt[:, 0]) <= 0)
    def _recover():
        m_t = jnp.where(any_in, -5.0, jnp.max(m_a, axis=1, keepdims=True))

        def rchunk(i, st):
            i_r, q_r = st
            sl = pl.ds(i * _CW, _CW)
            cos = jnp.clip(cos_scr[:, sl], -0.999999, 0.999999)
            q = (o2 + rhs4_ref[3:4, sl]) - 2.0 * q_scr[:, sl]
            u = jnp.logical_and(cos == m_t, i_r == _K)
            i_r = jnp.where(u, lane + i * _CW, i_r)
            q_r = jnp.where(u, q, q_r)
            return (i_r, q_r)

        i_r, q_r = jax.lax.fori_loop(0, _K // _CW, rchunk, (sent, zf))
        idx_r = jnp.min(i_r, axis=1, keepdims=True)
        q_rs = jnp.max(jnp.where(i_r == idx_r, q_r, -1.0), axis=1,
                       keepdims=True)
        m2 = jnp.where(any_in, m, m_t)
        idx2 = jnp.where(any_in, idx, idx_r)
        q2 = jnp.where(any_in, qsel, q_rs)
        angle2 = 2.0 * jnp.arctan2(jnp.sqrt(1.0 - m2 * m2), 1.0 + m2)
        dist2 = jnp.sqrt(jnp.maximum(q2, 0.0) + 1e-12)
        out_ref[...] = jnp.concatenate([angle2, dist2, tr9], axis=1)
        idx_ref[...] = idx2


def kernel(candidate_rotations, candidate_translations, target_pose):
    craw3 = candidate_rotations[:, :, 2].T
    ct3 = candidate_translations.T
    tpf = target_pose.reshape(_B, 16)

    cdirn, rhs4 = pl.pallas_call(
        _prep_kernel,
        out_shape=[
            jax.ShapeDtypeStruct((3, _K), jnp.float32),
            jax.ShapeDtypeStruct((4, _K), jnp.float32),
        ],
    )(craw3, ct3)

    out_f, idx, cnt = pl.pallas_call(
        _main_kernel,
        grid=(_B // _BB,),
        in_specs=[
            pl.BlockSpec((3, _K), lambda b: (0, 0)),
            pl.BlockSpec((4, _K), lambda b: (0, 0)),
            pl.BlockSpec((_BB, 16), lambda b: (b, 0)),
        ],
        out_specs=[
            pl.BlockSpec((_BB, 11), lambda b: (b, 0)),
            pl.BlockSpec((_BB, 1), lambda b: (b, 0)),
            pl.BlockSpec((_BB, 1), lambda b: (b, 0)),
        ],
        out_shape=[
            jax.ShapeDtypeStruct((_B, 11), jnp.float32),
            jax.ShapeDtypeStruct((_B, 1), jnp.int32),
            jax.ShapeDtypeStruct((_B, 1), jnp.int32),
        ],
        scratch_shapes=[
            pltpu.VMEM((_BB, _K), jnp.float32),
            pltpu.VMEM((_BB, _K), jnp.float32),
        ],
        compiler_params=pltpu.CompilerParams(
            dimension_semantics=("parallel",),
        ),
    )(cdirn, rhs4, tpf)
    return out_f, idx[:, 0], cnt[:, 0]


# speedup vs baseline: 1.5730x; 1.0307x over previous
import jax
import jax.numpy as jnp
from jax.experimental import pallas as pl
from jax.experimental.pallas import tpu as pltpu

_Q_THRESH = float.fromhex("0x1.47ae16p-1")
_B = 1024
_K = 16384
_BB = 64
_CW = 128


def _prep_kernel(craw_ref, ct_ref, cdirn_ref, rhs5_ref):
    craw = craw_ref[...]
    cn = jnp.sqrt(jnp.sum(craw * craw, axis=0, keepdims=True))
    cdirn_ref[...] = craw / (cn + 1e-8)
    ct = ct_ref[...]
    rhs5_ref[0:3, :] = ct
    rhs5_ref[3:4, :] = jnp.sum(ct * ct, axis=0, keepdims=True)


def _main_kernel(cdirn_ref, rhs4_ref, tpf_ref, out_ref, idx_ref, cnt_ref,
                 cos_scr, q_scr):
    tpf = tpf_ref[...]

    tdr = jnp.concatenate([tpf[:, 2:3], tpf[:, 6:7], tpf[:, 10:11]], axis=1)
    tn = jnp.sqrt(jnp.sum(tdr * tdr, axis=1, keepdims=True))
    tdir = tdr / (tn + 1e-8)
    torig = jnp.concatenate([tpf[:, 3:4], tpf[:, 7:8], tpf[:, 11:12]], axis=1)
    o2 = jnp.sum(torig * torig, axis=1, keepdims=True)

    cos_scr[...] = jax.lax.dot_general(
        tdir, cdirn_ref[...], (((1,), (0,)), ((), ())),
        preferred_element_type=jnp.float32)
    q_scr[...] = jax.lax.dot_general(
        torig, rhs4_ref[0:3, :], (((1,), (0,)), ((), ())),
        preferred_element_type=jnp.float32)

    lane = jax.lax.broadcasted_iota(jnp.int32, (_BB, _CW), 1)
    neg4 = jnp.full((_BB, _CW), -4.0, jnp.float32)
    sent = jnp.full((_BB, _CW), _K, jnp.int32)
    zf = jnp.zeros((_BB, _CW), jnp.float32)
    zi = jnp.zeros((_BB, _CW), jnp.int32)

    def chunk(i, st):
        m_i, i_i, q_i, m_a, cnt = st
        sl = pl.ds(i * _CW, _CW)
        cos = jnp.clip(cos_scr[:, sl], -0.999999, 0.999999)
        q = (o2 + rhs4_ref[3:4, sl]) - 2.0 * q_scr[:, sl]
        inr = q <= _Q_THRESH
        cnt = cnt + jnp.where(inr, 1, 0)
        idxv = lane + i * _CW
        eff = jnp.where(inr, cos, -3.0)
        u = eff > m_i
        m_i = jnp.where(u, eff, m_i)
        i_i = jnp.where(u, idxv, i_i)
        q_i = jnp.where(u, q, q_i)
        m_a = jnp.maximum(m_a, cos)
        return (m_i, i_i, q_i, m_a, cnt)

    m_i, i_i, q_i, m_a, cntl = jax.lax.fori_loop(
        0, _K // _CW, chunk, (neg4, sent, zf, neg4, zi), unroll=16)

    cnt = jnp.sum(cntl, axis=1, keepdims=True)
    any_in = cnt > 0
    m = jnp.max(m_i, axis=1, keepdims=True)
    idx = jnp.min(jnp.where(m_i == m, i_i, _K), axis=1, keepdims=True)
    qsel = jnp.max(jnp.where(i_i == idx, q_i, -1.0), axis=1, keepdims=True)

    angle = 2.0 * jnp.arctan2(jnp.sqrt(1.0 - m * m), 1.0 + m)
    dist = jnp.sqrt(jnp.maximum(qsel, 0.0) + 1e-12)

    tr9 = jnp.concatenate([tpf[:, 0:3], tpf[:, 4:7], tpf[:, 8:11]], axis=1)
    out_ref[...] = jnp.concatenate([angle, dist, tr9], axis=1)
    idx_ref[...] = idx
    cnt_ref[...] = cnt

    @pl.when(jnp.min(cnt[:, 0]) <= 0)
    def _recover():
        m_t = jnp.where(any_in, -5.0, jnp.max(m_a, axis=1, keepdims=True))

        def rchunk(i, st):
            i_r, q_r = st
            sl = pl.ds(i * _CW, _CW)
            cos = jnp.clip(cos_scr[:, sl], -0.999999, 0.999999)
            q = (o2 + rhs4_ref[3:4, sl]) - 2.0 * q_scr[:, sl]
            u = jnp.logical_and(cos == m_t, i_r == _K)
            i_r = jnp.where(u, lane + i * _CW, i_r)
            q_r = jnp.where(u, q, q_r)
            return (i_r, q_r)

        i_r, q_r = jax.lax.fori_loop(0, _K // _CW, rchunk, (sent, zf))
        idx_r = jnp.min(i_r, axis=1, keepdims=True)
        q_rs = jnp.max(jnp.where(i_r == idx_r, q_r, -1.0), axis=1,
                       keepdims=True)
        m2 = jnp.where(any_in, m, m_t)
        idx2 = jnp.where(any_in, idx, idx_r)
        q2 = jnp.where(any_in, qsel, q_rs)
        angle2 = 2.0 * jnp.arctan2(jnp.sqrt(1.0 - m2 * m2), 1.0 + m2)
        dist2 = jnp.sqrt(jnp.maximum(q2, 0.0) + 1e-12)
        out_ref[...] = jnp.concatenate([angle2, dist2, tr9], axis=1)
        idx_ref[...] = idx2


def kernel(candidate_rotations, candidate_translations, target_pose):
    craw3 = candidate_rotations[:, :, 2].T
    ct3 = candidate_translations.T
    tpf = target_pose.reshape(_B, 16)

    cdirn, rhs4 = pl.pallas_call(
        _prep_kernel,
        out_shape=[
            jax.ShapeDtypeStruct((3, _K), jnp.float32),
            jax.ShapeDtypeStruct((4, _K), jnp.float32),
        ],
    )(craw3, ct3)

    out_f, idx, cnt = pl.pallas_call(
        _main_kernel,
        grid=(_B // _BB,),
        in_specs=[
            pl.BlockSpec((3, _K), lambda b: (0, 0)),
            pl.BlockSpec((4, _K), lambda b: (0, 0)),
            pl.BlockSpec((_BB, 16), lambda b: (b, 0)),
        ],
        out_specs=[
            pl.BlockSpec((_BB, 11), lambda b: (b, 0)),
            pl.BlockSpec((_BB, 1), lambda b: (b, 0)),
            pl.BlockSpec((_BB, 1), lambda b: (b, 0)),
        ],
        out_shape=[
            jax.ShapeDtypeStruct((_B, 11), jnp.float32),
            jax.ShapeDtypeStruct((_B, 1), jnp.int32),
            jax.ShapeDtypeStruct((_B, 1), jnp.int32),
        ],
        scratch_shapes=[
            pltpu.VMEM((_BB, _K), jnp.float32),
            pltpu.VMEM((_BB, _K), jnp.float32),
        ],
        compiler_params=pltpu.CompilerParams(
            dimension_semantics=("parallel",),
        ),
    )(cdirn, rhs4, tpf)
    return out_f, idx[:, 0], cnt[:, 0]


# speedup vs baseline: 1.5864x; 1.0085x over previous
import jax
import jax.numpy as jnp
from jax.experimental import pallas as pl
from jax.experimental.pallas import tpu as pltpu

_Q_THRESH = float.fromhex("0x1.47ae16p-1")
_B = 1024
_K = 16384
_BB = 64
_CW = 128


def _prep_kernel(craw_ref, ct_ref, cdirn_ref, rhs5_ref):
    craw = craw_ref[...]
    cn = jnp.sqrt(jnp.sum(craw * craw, axis=0, keepdims=True))
    cdirn_ref[...] = craw / (cn + 1e-8)
    ct = ct_ref[...]
    rhs5_ref[0:3, :] = ct
    rhs5_ref[3:4, :] = jnp.sum(ct * ct, axis=0, keepdims=True)


def _main_kernel(cdirn_ref, rhs4_ref, tpf_ref, out_ref, idx_ref, cnt_ref,
                 cos_scr, q_scr):
    tpf = tpf_ref[...]

    tdr = jnp.concatenate([tpf[:, 2:3], tpf[:, 6:7], tpf[:, 10:11]], axis=1)
    tn = jnp.sqrt(jnp.sum(tdr * tdr, axis=1, keepdims=True))
    tdir = tdr / (tn + 1e-8)
    torig = jnp.concatenate([tpf[:, 3:4], tpf[:, 7:8], tpf[:, 11:12]], axis=1)
    o2 = jnp.sum(torig * torig, axis=1, keepdims=True)

    cos_scr[...] = jax.lax.dot_general(
        tdir, cdirn_ref[...], (((1,), (0,)), ((), ())),
        preferred_element_type=jnp.float32)
    q_scr[...] = jax.lax.dot_general(
        torig, rhs4_ref[0:3, :], (((1,), (0,)), ((), ())),
        preferred_element_type=jnp.float32)

    lane = jax.lax.broadcasted_iota(jnp.int32, (_BB, _CW), 1)
    neg4 = jnp.full((_BB, _CW), -4.0, jnp.float32)
    sent = jnp.full((_BB, _CW), _K, jnp.int32)
    zf = jnp.zeros((_BB, _CW), jnp.float32)
    zi = jnp.zeros((_BB, _CW), jnp.int32)

    def chunk(i, st):
        m_i, i_i, q_i, m_a, cnt = st
        sl = pl.ds(i * _CW, _CW)
        cos = jnp.clip(cos_scr[:, sl], -0.999999, 0.999999)
        q = (o2 + rhs4_ref[3:4, sl]) - 2.0 * q_scr[:, sl]
        inr = q <= _Q_THRESH
        cnt = cnt + jnp.where(inr, 1, 0)
        idxv = lane + i * _CW
        eff = jnp.where(inr, cos, -3.0)
        u = eff > m_i
        m_i = jnp.where(u, eff, m_i)
        i_i = jnp.where(u, idxv, i_i)
        q_i = jnp.where(u, q, q_i)
        m_a = jnp.maximum(m_a, cos)
        return (m_i, i_i, q_i, m_a, cnt)

    m_i, i_i, q_i, m_a, cntl = jax.lax.fori_loop(
        0, _K // _CW, chunk, (neg4, sent, zf, neg4, zi), unroll=32)

    cnt = jnp.sum(cntl, axis=1, keepdims=True)
    any_in = cnt > 0
    m = jnp.max(m_i, axis=1, keepdims=True)
    idx = jnp.min(jnp.where(m_i == m, i_i, _K), axis=1, keepdims=True)
    qsel = jnp.max(jnp.where(i_i == idx, q_i, -1.0), axis=1, keepdims=True)

    angle = 2.0 * jnp.arctan2(jnp.sqrt(1.0 - m * m), 1.0 + m)
    dist = jnp.sqrt(jnp.maximum(qsel, 0.0) + 1e-12)

    tr9 = jnp.concatenate([tpf[:, 0:3], tpf[:, 4:7], tpf[:, 8:11]], axis=1)
    out_ref[...] = jnp.concatenate([angle, dist, tr9], axis=1)
    idx_ref[...] = idx
    cnt_ref[...] = cnt

    @pl.when(jnp.min(cnt[:, 0]) <= 0)
    def _recover():
        m_t = jnp.where(any_in, -5.0, jnp.max(m_a, axis=1, keepdims=True))

        def rchunk(i, st):
            i_r, q_r = st
            sl = pl.ds(i * _CW, _CW)
            cos = jnp.clip(cos_scr[:, sl], -0.999999, 0.999999)
            q = (o2 + rhs4_ref[3:4, sl]) - 2.0 * q_scr[:, sl]
            u = jnp.logical_and(cos == m_t, i_r == _K)
            i_r = jnp.where(u, lane + i * _CW, i_r)
            q_r = jnp.where(u, q, q_r)
            return (i_r, q_r)

        i_r, q_r = jax.lax.fori_loop(0, _K // _CW, rchunk, (sent, zf))
        idx_r = jnp.min(i_r, axis=1, keepdims=True)
        q_rs = jnp.max(jnp.where(i_r == idx_r, q_r, -1.0), axis=1,
                       keepdims=True)
        m2 = jnp.where(any_in, m, m_t)
        idx2 = jnp.where(any_in, idx, idx_r)
        q2 = jnp.where(any_in, qsel, q_rs)
        angle2 = 2.0 * jnp.arctan2(jnp.sqrt(1.0 - m2 * m2), 1.0 + m2)
        dist2 = jnp.sqrt(jnp.maximum(q2, 0.0) + 1e-12)
        out_ref[...] = jnp.concatenate([angle2, dist2, tr9], axis=1)
        idx_ref[...] = idx2


def kernel(candidate_rotations, candidate_translations, target_pose):
    craw3 = candidate_rotations[:, :, 2].T
    ct3 = candidate_translations.T
    tpf = target_pose.reshape(_B, 16)

    cdirn, rhs4 = pl.pallas_call(
        _prep_kernel,
        out_shape=[
            jax.ShapeDtypeStruct((3, _K), jnp.float32),
            jax.ShapeDtypeStruct((4, _K), jnp.float32),
        ],
    )(craw3, ct3)

    out_f, idx, cnt = pl.pallas_call(
        _main_kernel,
        grid=(_B // _BB,),
        in_specs=[
            pl.BlockSpec((3, _K), lambda b: (0, 0)),
            pl.BlockSpec((4, _K), lambda b: (0, 0)),
            pl.BlockSpec((_BB, 16), lambda b: (b, 0)),
        ],
        out_specs=[
            pl.BlockSpec((_BB, 11), lambda b: (b, 0)),
            pl.BlockSpec((_BB, 1), lambda b: (b, 0)),
            pl.BlockSpec((_BB, 1), lambda b: (b, 0)),
        ],
        out_shape=[
            jax.ShapeDtypeStruct((_B, 11), jnp.float32),
            jax.ShapeDtypeStruct((_B, 1), jnp.int32),
            jax.ShapeDtypeStruct((_B, 1), jnp.int32),
        ],
        scratch_shapes=[
            pltpu.VMEM((_BB, _K), jnp.float32),
            pltpu.VMEM((_BB, _K), jnp.float32),
        ],
        compiler_params=pltpu.CompilerParams(
            dimension_semantics=("parallel",),
        ),
    )(cdirn, rhs4, tpf)
    return out_f, idx[:, 0], cnt[:, 0]


# speedup vs baseline: 2.0317x; 1.2807x over previous
import jax
import jax.numpy as jnp
from jax.experimental import pallas as pl
from jax.experimental.pallas import tpu as pltpu

_Q_THRESH = float.fromhex("0x1.47ae16p-1")
_B = 1024
_K = 16384
_BB = 64
_CW = 128


def _prep_kernel(craw_ref, ct_ref, cdirn_ref, rhs5_ref):
    craw = craw_ref[...]
    cn = jnp.sqrt(jnp.sum(craw * craw, axis=0, keepdims=True))
    cdirn_ref[...] = craw / (cn + 1e-8)
    ct = ct_ref[...]
    rhs5_ref[0:3, :] = ct
    rhs5_ref[3:4, :] = jnp.sum(ct * ct, axis=0, keepdims=True)


def _main_kernel(cdirn_ref, rhs4_ref, tpf_ref, out_ref, idx_ref, cnt_ref,
                 cos_scr, q_scr):
    tpf = tpf_ref[...]

    tdr = jnp.concatenate([tpf[:, 2:3], tpf[:, 6:7], tpf[:, 10:11]], axis=1)
    tn = jnp.sqrt(jnp.sum(tdr * tdr, axis=1, keepdims=True))
    tdir = tdr / (tn + 1e-8)
    torig = jnp.concatenate([tpf[:, 3:4], tpf[:, 7:8], tpf[:, 11:12]], axis=1)
    o2 = jnp.sum(torig * torig, axis=1, keepdims=True)

    lane = jax.lax.broadcasted_iota(jnp.int32, (_BB, _CW), 1)
    sent = jnp.full((_BB, _CW), _K, jnp.int32)
    zf = jnp.zeros((_BB, _CW), jnp.float32)

    def chunk(i, st):
        m_i, i_i, q_i, m_a, cnt = st
        sl = pl.ds(i * _CW, _CW)
        cos = jnp.clip(cos_scr[:, sl], -0.999999, 0.999999)
        q = (o2 + rhs4_ref[3:4, sl]) - 2.0 * q_scr[:, sl]
        inr = q <= _Q_THRESH
        cnt = cnt + jnp.where(inr, 1, 0)
        idxv = lane + i * _CW
        eff = jnp.where(inr, cos, -3.0)
        u = eff > m_i
        m_i = jnp.where(u, eff, m_i)
        i_i = jnp.where(u, idxv, i_i)
        q_i = jnp.where(u, q, q_i)
        m_a = jnp.maximum(m_a, cos)
        return (m_i, i_i, q_i, m_a, cnt)

    _NS = 8
    _SW = _K // _NS
    st = (jnp.full((_BB, _CW), -4.0, jnp.float32),
          jnp.full((_BB, _CW), _K, jnp.int32),
          jnp.zeros((_BB, _CW), jnp.float32),
          jnp.full((_BB, _CW), -4.0, jnp.float32),
          jnp.zeros((_BB, _CW), jnp.int32))
    for s in range(_NS + 1):
        if s < _NS:
            ss = slice(s * _SW, (s + 1) * _SW)
            cos_scr[:, ss] = jax.lax.dot_general(
                tdir, cdirn_ref[:, ss], (((1,), (0,)), ((), ())),
                preferred_element_type=jnp.float32)
            q_scr[:, ss] = jax.lax.dot_general(
                torig, rhs4_ref[0:3, ss], (((1,), (0,)), ((), ())),
                preferred_element_type=jnp.float32)
        if s >= 1:
            base = (s - 1) * (_SW // _CW)
            st = jax.lax.fori_loop(base, base + _SW // _CW, chunk, st,
                                   unroll=_SW // _CW)
    m_i, i_i, q_i, m_a, cntl = st

    cnt = jnp.sum(cntl, axis=1, keepdims=True)
    any_in = cnt > 0
    m = jnp.max(m_i, axis=1, keepdims=True)
    idx = jnp.min(jnp.where(m_i == m, i_i, _K), axis=1, keepdims=True)
    qsel = jnp.max(jnp.where(i_i == idx, q_i, -1.0), axis=1, keepdims=True)

    angle = 2.0 * jnp.arctan2(jnp.sqrt(1.0 - m * m), 1.0 + m)
    dist = jnp.sqrt(jnp.maximum(qsel, 0.0) + 1e-12)

    tr9 = jnp.concatenate([tpf[:, 0:3], tpf[:, 4:7], tpf[:, 8:11]], axis=1)
    out_ref[...] = jnp.concatenate([angle, dist, tr9], axis=1)
    idx_ref[...] = idx
    cnt_ref[...] = cnt

    @pl.when(jnp.min(cnt[:, 0]) <= 0)
    def _recover():
        m_t = jnp.where(any_in, -5.0, jnp.max(m_a, axis=1, keepdims=True))

        def rchunk(i, st):
            i_r, q_r = st
            sl = pl.ds(i * _CW, _CW)
            cos = jnp.clip(cos_scr[:, sl], -0.999999, 0.999999)
            q = (o2 + rhs4_ref[3:4, sl]) - 2.0 * q_scr[:, sl]
            u = jnp.logical_and(cos == m_t, i_r == _K)
            i_r = jnp.where(u, lane + i * _CW, i_r)
            q_r = jnp.where(u, q, q_r)
            return (i_r, q_r)

        i_r, q_r = jax.lax.fori_loop(0, _K // _CW, rchunk, (sent, zf))
        idx_r = jnp.min(i_r, axis=1, keepdims=True)
        q_rs = jnp.max(jnp.where(i_r == idx_r, q_r, -1.0), axis=1,
                       keepdims=True)
        m2 = jnp.where(any_in, m, m_t)
        idx2 = jnp.where(any_in, idx, idx_r)
        q2 = jnp.where(any_in, qsel, q_rs)
        angle2 = 2.0 * jnp.arctan2(jnp.sqrt(1.0 - m2 * m2), 1.0 + m2)
        dist2 = jnp.sqrt(jnp.maximum(q2, 0.0) + 1e-12)
        out_ref[...] = jnp.concatenate([angle2, dist2, tr9], axis=1)
        idx_ref[...] = idx2


def kernel(candidate_rotations, candidate_translations, target_pose):
    craw3 = candidate_rotations[:, :, 2].T
    ct3 = candidate_translations.T
    tpf = target_pose.reshape(_B, 16)

    cdirn, rhs4 = pl.pallas_call(
        _prep_kernel,
        out_shape=[
            jax.ShapeDtypeStruct((3, _K), jnp.float32),
            jax.ShapeDtypeStruct((4, _K), jnp.float32),
        ],
    )(craw3, ct3)

    out_f, idx, cnt = pl.pallas_call(
        _main_kernel,
        grid=(_B // _BB,),
        in_specs=[
            pl.BlockSpec((3, _K), lambda b: (0, 0)),
            pl.BlockSpec((4, _K), lambda b: (0, 0)),
            pl.BlockSpec((_BB, 16), lambda b: (b, 0)),
        ],
        out_specs=[
            pl.BlockSpec((_BB, 11), lambda b: (b, 0)),
            pl.BlockSpec((_BB, 1), lambda b: (b, 0)),
            pl.BlockSpec((_BB, 1), lambda b: (b, 0)),
        ],
        out_shape=[
            jax.ShapeDtypeStruct((_B, 11), jnp.float32),
            jax.ShapeDtypeStruct((_B, 1), jnp.int32),
            jax.ShapeDtypeStruct((_B, 1), jnp.int32),
        ],
        scratch_shapes=[
            pltpu.VMEM((_BB, _K), jnp.float32),
            pltpu.VMEM((_BB, _K), jnp.float32),
        ],
        compiler_params=pltpu.CompilerParams(
            dimension_semantics=("parallel",),
        ),
    )(cdirn, rhs4, tpf)
    return out_f, idx[:, 0], cnt[:, 0]


# speedup vs baseline: 2.0803x; 1.0240x over previous
import jax
import jax.numpy as jnp
from jax.experimental import pallas as pl
from jax.experimental.pallas import tpu as pltpu

_Q_THRESH = float.fromhex("0x1.47ae16p-1")
_B = 1024
_K = 16384
_BB = 128
_CW = 128


def _prep_kernel(craw_ref, ct_ref, cdirn_ref, rhs5_ref):
    craw = craw_ref[...]
    cn = jnp.sqrt(jnp.sum(craw * craw, axis=0, keepdims=True))
    cdirn_ref[...] = craw / (cn + 1e-8)
    ct = ct_ref[...]
    rhs5_ref[0:3, :] = ct
    rhs5_ref[3:4, :] = jnp.sum(ct * ct, axis=0, keepdims=True)


def _main_kernel(cdirn_ref, rhs4_ref, tpf_ref, out_ref, idx_ref, cnt_ref,
                 cos_scr, q_scr):
    tpf = tpf_ref[...]

    tdr = jnp.concatenate([tpf[:, 2:3], tpf[:, 6:7], tpf[:, 10:11]], axis=1)
    tn = jnp.sqrt(jnp.sum(tdr * tdr, axis=1, keepdims=True))
    tdir = tdr / (tn + 1e-8)
    torig = jnp.concatenate([tpf[:, 3:4], tpf[:, 7:8], tpf[:, 11:12]], axis=1)
    o2 = jnp.sum(torig * torig, axis=1, keepdims=True)

    lane = jax.lax.broadcasted_iota(jnp.int32, (_BB, _CW), 1)
    sent = jnp.full((_BB, _CW), _K, jnp.int32)
    zf = jnp.zeros((_BB, _CW), jnp.float32)

    def chunk(i, st):
        m_i, i_i, q_i, m_a, cnt = st
        sl = pl.ds(i * _CW, _CW)
        cos = jnp.clip(cos_scr[:, sl], -0.999999, 0.999999)
        q = (o2 + rhs4_ref[3:4, sl]) - 2.0 * q_scr[:, sl]
        inr = q <= _Q_THRESH
        cnt = cnt + jnp.where(inr, 1, 0)
        idxv = lane + i * _CW
        eff = jnp.where(inr, cos, -3.0)
        u = eff > m_i
        m_i = jnp.where(u, eff, m_i)
        i_i = jnp.where(u, idxv, i_i)
        q_i = jnp.where(u, q, q_i)
        m_a = jnp.maximum(m_a, cos)
        return (m_i, i_i, q_i, m_a, cnt)

    _NS = 8
    _SW = _K // _NS
    st = (jnp.full((_BB, _CW), -4.0, jnp.float32),
          jnp.full((_BB, _CW), _K, jnp.int32),
          jnp.zeros((_BB, _CW), jnp.float32),
          jnp.full((_BB, _CW), -4.0, jnp.float32),
          jnp.zeros((_BB, _CW), jnp.int32))
    for s in range(_NS + 1):
        if s < _NS:
            ss = slice(s * _SW, (s + 1) * _SW)
            cos_scr[:, ss] = jax.lax.dot_general(
                tdir, cdirn_ref[:, ss], (((1,), (0,)), ((), ())),
                preferred_element_type=jnp.float32)
            q_scr[:, ss] = jax.lax.dot_general(
                torig, rhs4_ref[0:3, ss], (((1,), (0,)), ((), ())),
                preferred_element_type=jnp.float32)
        if s >= 1:
            base = (s - 1) * (_SW // _CW)
            st = jax.lax.fori_loop(base, base + _SW // _CW, chunk, st,
                                   unroll=_SW // _CW)
    m_i, i_i, q_i, m_a, cntl = st

    cnt = jnp.sum(cntl, axis=1, keepdims=True)
    any_in = cnt > 0
    m = jnp.max(m_i, axis=1, keepdims=True)
    idx = jnp.min(jnp.where(m_i == m, i_i, _K), axis=1, keepdims=True)
    qsel = jnp.max(jnp.where(i_i == idx, q_i, -1.0), axis=1, keepdims=True)

    angle = 2.0 * jnp.arctan2(jnp.sqrt(1.0 - m * m), 1.0 + m)
    dist = jnp.sqrt(jnp.maximum(qsel, 0.0) + 1e-12)

    tr9 = jnp.concatenate([tpf[:, 0:3], tpf[:, 4:7], tpf[:, 8:11]], axis=1)
    out_ref[...] = jnp.concatenate([angle, dist, tr9], axis=1)
    idx_ref[...] = idx
    cnt_ref[...] = cnt

    @pl.when(jnp.min(cnt[:, 0]) <= 0)
    def _recover():
        m_t = jnp.where(any_in, -5.0, jnp.max(m_a, axis=1, keepdims=True))

        def rchunk(i, st):
            i_r, q_r = st
            sl = pl.ds(i * _CW, _CW)
            cos = jnp.clip(cos_scr[:, sl], -0.999999, 0.999999)
            q = (o2 + rhs4_ref[3:4, sl]) - 2.0 * q_scr[:, sl]
            u = jnp.logical_and(cos == m_t, i_r == _K)
            i_r = jnp.where(u, lane + i * _CW, i_r)
            q_r = jnp.where(u, q, q_r)
            return (i_r, q_r)

        i_r, q_r = jax.lax.fori_loop(0, _K // _CW, rchunk, (sent, zf))
        idx_r = jnp.min(i_r, axis=1, keepdims=True)
        q_rs = jnp.max(jnp.where(i_r == idx_r, q_r, -1.0), axis=1,
                       keepdims=True)
        m2 = jnp.where(any_in, m, m_t)
        idx2 = jnp.where(any_in, idx, idx_r)
        q2 = jnp.where(any_in, qsel, q_rs)
        angle2 = 2.0 * jnp.arctan2(jnp.sqrt(1.0 - m2 * m2), 1.0 + m2)
        dist2 = jnp.sqrt(jnp.maximum(q2, 0.0) + 1e-12)
        out_ref[...] = jnp.concatenate([angle2, dist2, tr9], axis=1)
        idx_ref[...] = idx2


def kernel(candidate_rotations, candidate_translations, target_pose):
    craw3 = candidate_rotations[:, :, 2].T
    ct3 = candidate_translations.T
    tpf = target_pose.reshape(_B, 16)

    cdirn, rhs4 = pl.pallas_call(
        _prep_kernel,
        out_shape=[
            jax.ShapeDtypeStruct((3, _K), jnp.float32),
            jax.ShapeDtypeStruct((4, _K), jnp.float32),
        ],
    )(craw3, ct3)

    out_f, idx, cnt = pl.pallas_call(
        _main_kernel,
        grid=(_B // _BB,),
        in_specs=[
            pl.BlockSpec((3, _K), lambda b: (0, 0)),
            pl.BlockSpec((4, _K), lambda b: (0, 0)),
            pl.BlockSpec((_BB, 16), lambda b: (b, 0)),
        ],
        out_specs=[
            pl.BlockSpec((_BB, 11), lambda b: (b, 0)),
            pl.BlockSpec((_BB, 1), lambda b: (b, 0)),
            pl.BlockSpec((_BB, 1), lambda b: (b, 0)),
        ],
        out_shape=[
            jax.ShapeDtypeStruct((_B, 11), jnp.float32),
            jax.ShapeDtypeStruct((_B, 1), jnp.int32),
            jax.ShapeDtypeStruct((_B, 1), jnp.int32),
        ],
        scratch_shapes=[
            pltpu.VMEM((_BB, _K), jnp.float32),
            pltpu.VMEM((_BB, _K), jnp.float32),
        ],
        compiler_params=pltpu.CompilerParams(
            dimension_semantics=("parallel",),
        ),
    )(cdirn, rhs4, tpf)
    return out_f, idx[:, 0], cnt[:, 0]


# speedup vs baseline: 2.1152x; 1.0168x over previous
import jax
import jax.numpy as jnp
from jax.experimental import pallas as pl
from jax.experimental.pallas import tpu as pltpu

_Q_THRESH = float.fromhex("0x1.47ae16p-1")
_B = 1024
_K = 16384
_BB = 128
_CW = 128


def _prep_kernel(craw_ref, ct_ref, cdirn_ref, rhs5_ref):
    craw = craw_ref[...]
    cn = jnp.sqrt(jnp.sum(craw * craw, axis=0, keepdims=True))
    cdirn_ref[...] = craw / (cn + 1e-8)
    ct = ct_ref[...]
    rhs5_ref[0:3, :] = ct
    rhs5_ref[3:4, :] = jnp.sum(ct * ct, axis=0, keepdims=True)


def _main_kernel(cdirn_ref, rhs4_ref, tpf_ref, out_ref, idx_ref, cnt_ref,
                 cos_scr, q_scr):
    tpf = tpf_ref[...]

    tdr = jnp.concatenate([tpf[:, 2:3], tpf[:, 6:7], tpf[:, 10:11]], axis=1)
    tn = jnp.sqrt(jnp.sum(tdr * tdr, axis=1, keepdims=True))
    tdir = tdr / (tn + 1e-8)
    torig = jnp.concatenate([tpf[:, 3:4], tpf[:, 7:8], tpf[:, 11:12]], axis=1)
    o2 = jnp.sum(torig * torig, axis=1, keepdims=True)

    lane = jax.lax.broadcasted_iota(jnp.int32, (_BB, _CW), 1)
    sent = jnp.full((_BB, _CW), _K, jnp.int32)
    zf = jnp.zeros((_BB, _CW), jnp.float32)

    def chunk(i, st):
        m_i, i_i, q_i, cnt = st
        sl = pl.ds(i * _CW, _CW)
        cos = jnp.clip(cos_scr[:, sl], -0.999999, 0.999999)
        q = (o2 + rhs4_ref[3:4, sl]) - 2.0 * q_scr[:, sl]
        inr = q <= _Q_THRESH
        cnt = cnt + jnp.where(inr, 1, 0)
        idxv = lane + i * _CW
        eff = jnp.where(inr, cos, -3.0)
        u = eff > m_i
        m_i = jnp.where(u, eff, m_i)
        i_i = jnp.where(u, idxv, i_i)
        q_i = jnp.where(u, q, q_i)
        return (m_i, i_i, q_i, cnt)

    _NS = 8
    _SW = _K // _NS
    st = (jnp.full((_BB, _CW), -4.0, jnp.float32),
          jnp.full((_BB, _CW), _K, jnp.int32),
          jnp.zeros((_BB, _CW), jnp.float32),
          jnp.zeros((_BB, _CW), jnp.int32))
    for s in range(_NS + 1):
        if s < _NS:
            ss = slice(s * _SW, (s + 1) * _SW)
            cos_scr[:, ss] = jax.lax.dot_general(
                tdir, cdirn_ref[:, ss], (((1,), (0,)), ((), ())),
                preferred_element_type=jnp.float32)
            q_scr[:, ss] = jax.lax.dot_general(
                torig, rhs4_ref[0:3, ss], (((1,), (0,)), ((), ())),
                preferred_element_type=jnp.float32)
        if s >= 1:
            base = (s - 1) * (_SW // _CW)
            st = jax.lax.fori_loop(base, base + _SW // _CW, chunk, st,
                                   unroll=_SW // _CW)
    m_i, i_i, q_i, cntl = st

    cnt = jnp.sum(cntl, axis=1, keepdims=True)
    any_in = cnt > 0
    m = jnp.max(m_i, axis=1, keepdims=True)
    idx = jnp.min(jnp.where(m_i == m, i_i, _K), axis=1, keepdims=True)
    qsel = jnp.max(jnp.where(i_i == idx, q_i, -1.0), axis=1, keepdims=True)

    angle = 2.0 * jnp.arctan2(jnp.sqrt(1.0 - m * m), 1.0 + m)
    dist = jnp.sqrt(jnp.maximum(qsel, 0.0) + 1e-12)

    tr9 = jnp.concatenate([tpf[:, 0:3], tpf[:, 4:7], tpf[:, 8:11]], axis=1)
    out_ref[...] = jnp.concatenate([angle, dist, tr9], axis=1)
    idx_ref[...] = idx
    cnt_ref[...] = cnt

    @pl.when(jnp.min(cnt[:, 0]) <= 0)
    def _recover():
        def mchunk(i, macc):
            sl = pl.ds(i * _CW, _CW)
            return jnp.maximum(
                macc, jnp.clip(cos_scr[:, sl], -0.999999, 0.999999))

        m_a = jax.lax.fori_loop(
            0, _K // _CW, mchunk, jnp.full((_BB, _CW), -4.0, jnp.float32))
        m_t = jnp.where(any_in, -5.0, jnp.max(m_a, axis=1, keepdims=True))

        def rchunk(i, st):
            i_r, q_r = st
            sl = pl.ds(i * _CW, _CW)
            cos = jnp.clip(cos_scr[:, sl], -0.999999, 0.999999)
            q = (o2 + rhs4_ref[3:4, sl]) - 2.0 * q_scr[:, sl]
            u = jnp.logical_and(cos == m_t, i_r == _K)
            i_r = jnp.where(u, lane + i * _CW, i_r)
            q_r = jnp.where(u, q, q_r)
            return (i_r, q_r)

        i_r, q_r = jax.lax.fori_loop(0, _K // _CW, rchunk, (sent, zf))
        idx_r = jnp.min(i_r, axis=1, keepdims=True)
        q_rs = jnp.max(jnp.where(i_r == idx_r, q_r, -1.0), axis=1,
                       keepdims=True)
        m2 = jnp.where(any_in, m, m_t)
        idx2 = jnp.where(any_in, idx, idx_r)
        q2 = jnp.where(any_in, qsel, q_rs)
        angle2 = 2.0 * jnp.arctan2(jnp.sqrt(1.0 - m2 * m2), 1.0 + m2)
        dist2 = jnp.sqrt(jnp.maximum(q2, 0.0) + 1e-12)
        out_ref[...] = jnp.concatenate([angle2, dist2, tr9], axis=1)
        idx_ref[...] = idx2


def kernel(candidate_rotations, candidate_translations, target_pose):
    craw3 = candidate_rotations[:, :, 2].T
    ct3 = candidate_translations.T
    tpf = target_pose.reshape(_B, 16)

    cdirn, rhs4 = pl.pallas_call(
        _prep_kernel,
        out_shape=[
            jax.ShapeDtypeStruct((3, _K), jnp.float32),
            jax.ShapeDtypeStruct((4, _K), jnp.float32),
        ],
    )(craw3, ct3)

    out_f, idx, cnt = pl.pallas_call(
        _main_kernel,
        grid=(_B // _BB,),
        in_specs=[
            pl.BlockSpec((3, _K), lambda b: (0, 0)),
            pl.BlockSpec((4, _K), lambda b: (0, 0)),
            pl.BlockSpec((_BB, 16), lambda b: (b, 0)),
        ],
        out_specs=[
            pl.BlockSpec((_BB, 11), lambda b: (b, 0)),
            pl.BlockSpec((_BB, 1), lambda b: (b, 0)),
            pl.BlockSpec((_BB, 1), lambda b: (b, 0)),
        ],
        out_shape=[
            jax.ShapeDtypeStruct((_B, 11), jnp.float32),
            jax.ShapeDtypeStruct((_B, 1), jnp.int32),
            jax.ShapeDtypeStruct((_B, 1), jnp.int32),
        ],
        scratch_shapes=[
            pltpu.VMEM((_BB, _K), jnp.float32),
            pltpu.VMEM((_BB, _K), jnp.float32),
        ],
        compiler_params=pltpu.CompilerParams(
            dimension_semantics=("parallel",),
        ),
    )(cdirn, rhs4, tpf)
    return out_f, idx[:, 0], cnt[:, 0]


# speedup vs baseline: 2.3148x; 1.0944x over previous
import jax
import jax.numpy as jnp
from jax.experimental import pallas as pl
from jax.experimental.pallas import tpu as pltpu

_Q_THRESH = float.fromhex("0x1.47ae16p-1")
_B = 1024
_K = 16384
_BB = 128
_CW = 128


def _prep_kernel(craw_ref, ct_ref, cdirn_ref, rhs5_ref):
    craw = craw_ref[...]
    cn = jnp.sqrt(jnp.sum(craw * craw, axis=0, keepdims=True))
    cdirn_ref[...] = craw / (cn + 1e-8)
    ct = ct_ref[...]
    rhs5_ref[0:3, :] = ct
    rhs5_ref[3:4, :] = jnp.sum(ct * ct, axis=0, keepdims=True)


def _main_kernel(cdirn_ref, rhs4_ref, tpf_ref, out_ref, idx_ref, cnt_ref,
                 cos_scr, q_scr):
    tpf = tpf_ref[...]

    tdr = jnp.concatenate([tpf[:, 2:3], tpf[:, 6:7], tpf[:, 10:11]], axis=1)
    tn = jnp.sqrt(jnp.sum(tdr * tdr, axis=1, keepdims=True))
    tdir = tdr / (tn + 1e-8)
    torig = jnp.concatenate([tpf[:, 3:4], tpf[:, 7:8], tpf[:, 11:12]], axis=1)
    o2 = jnp.sum(torig * torig, axis=1, keepdims=True)

    lane = jax.lax.broadcasted_iota(jnp.int32, (_BB, _CW), 1)
    sent = jnp.full((_BB, _CW), _K, jnp.int32)
    zf = jnp.zeros((_BB, _CW), jnp.float32)

    def chunk(i, st):
        m_i, i_i, q_i, cnt = st
        sl = pl.ds(i * _CW, _CW)
        cos = jnp.clip(cos_scr[:, sl], -0.999999, 0.999999)
        q = (o2 + rhs4_ref[3:4, sl]) - 2.0 * q_scr[:, sl]
        inr = q <= _Q_THRESH
        cnt = cnt + jnp.where(inr, 1, 0)
        idxv = lane + i * _CW
        eff = jnp.where(inr, cos, -3.0)
        u = eff > m_i
        m_i = jnp.where(u, eff, m_i)
        i_i = jnp.where(u, idxv, i_i)
        q_i = jnp.where(u, q, q_i)
        return (m_i, i_i, q_i, cnt)

    _NS = 16
    _SW = _K // _NS
    st = (jnp.full((_BB, _CW), -4.0, jnp.float32),
          jnp.full((_BB, _CW), _K, jnp.int32),
          jnp.zeros((_BB, _CW), jnp.float32),
          jnp.zeros((_BB, _CW), jnp.int32))
    for s in range(_NS + 1):
        if s < _NS:
            ss = slice(s * _SW, (s + 1) * _SW)
            cos_scr[:, ss] = jax.lax.dot_general(
                tdir, cdirn_ref[:, ss], (((1,), (0,)), ((), ())),
                preferred_element_type=jnp.float32)
            q_scr[:, ss] = jax.lax.dot_general(
                torig, rhs4_ref[0:3, ss], (((1,), (0,)), ((), ())),
                preferred_element_type=jnp.float32)
        if s >= 1:
            base = (s - 1) * (_SW // _CW)
            st = jax.lax.fori_loop(base, base + _SW // _CW, chunk, st,
                                   unroll=_SW // _CW)
    m_i, i_i, q_i, cntl = st

    cnt = jnp.sum(cntl, axis=1, keepdims=True)
    any_in = cnt > 0
    m = jnp.max(m_i, axis=1, keepdims=True)
    idx = jnp.min(jnp.where(m_i == m, i_i, _K), axis=1, keepdims=True)
    qsel = jnp.max(jnp.where(i_i == idx, q_i, -1.0), axis=1, keepdims=True)

    angle = 2.0 * jnp.arctan2(jnp.sqrt(1.0 - m * m), 1.0 + m)
    dist = jnp.sqrt(jnp.maximum(qsel, 0.0) + 1e-12)

    tr9 = jnp.concatenate([tpf[:, 0:3], tpf[:, 4:7], tpf[:, 8:11]], axis=1)
    out_ref[...] = jnp.concatenate([angle, dist, tr9], axis=1)
    idx_ref[...] = idx
    cnt_ref[...] = cnt

    @pl.when(jnp.min(cnt[:, 0]) <= 0)
    def _recover():
        def mchunk(i, macc):
            sl = pl.ds(i * _CW, _CW)
            return jnp.maximum(
                macc, jnp.clip(cos_scr[:, sl], -0.999999, 0.999999))

        m_a = jax.lax.fori_loop(
            0, _K // _CW, mchunk, jnp.full((_BB, _CW), -4.0, jnp.float32))
        m_t = jnp.where(any_in, -5.0, jnp.max(m_a, axis=1, keepdims=True))

        def rchunk(i, st):
            i_r, q_r = st
            sl = pl.ds(i * _CW, _CW)
            cos = jnp.clip(cos_scr[:, sl], -0.999999, 0.999999)
            q = (o2 + rhs4_ref[3:4, sl]) - 2.0 * q_scr[:, sl]
            u = jnp.logical_and(cos == m_t, i_r == _K)
            i_r = jnp.where(u, lane + i * _CW, i_r)
            q_r = jnp.where(u, q, q_r)
            return (i_r, q_r)

        i_r, q_r = jax.lax.fori_loop(0, _K // _CW, rchunk, (sent, zf))
        idx_r = jnp.min(i_r, axis=1, keepdims=True)
        q_rs = jnp.max(jnp.where(i_r == idx_r, q_r, -1.0), axis=1,
                       keepdims=True)
        m2 = jnp.where(any_in, m, m_t)
        idx2 = jnp.where(any_in, idx, idx_r)
        q2 = jnp.where(any_in, qsel, q_rs)
        angle2 = 2.0 * jnp.arctan2(jnp.sqrt(1.0 - m2 * m2), 1.0 + m2)
        dist2 = jnp.sqrt(jnp.maximum(q2, 0.0) + 1e-12)
        out_ref[...] = jnp.concatenate([angle2, dist2, tr9], axis=1)
        idx_ref[...] = idx2


def kernel(candidate_rotations, candidate_translations, target_pose):
    craw3 = candidate_rotations[:, :, 2].T
    ct3 = candidate_translations.T
    tpf = target_pose.reshape(_B, 16)

    cdirn, rhs4 = pl.pallas_call(
        _prep_kernel,
        out_shape=[
            jax.ShapeDtypeStruct((3, _K), jnp.float32),
            jax.ShapeDtypeStruct((4, _K), jnp.float32),
        ],
    )(craw3, ct3)

    out_f, idx, cnt = pl.pallas_call(
        _main_kernel,
        grid=(_B // _BB,),
        in_specs=[
            pl.BlockSpec((3, _K), lambda b: (0, 0)),
            pl.BlockSpec((4, _K), lambda b: (0, 0)),
            pl.BlockSpec((_BB, 16), lambda b: (b, 0)),
        ],
        out_specs=[
            pl.BlockSpec((_BB, 11), lambda b: (b, 0)),
            pl.BlockSpec((_BB, 1), lambda b: (b, 0)),
            pl.BlockSpec((_BB, 1), lambda b: (b, 0)),
        ],
        out_shape=[
            jax.ShapeDtypeStruct((_B, 11), jnp.float32),
            jax.ShapeDtypeStruct((_B, 1), jnp.int32),
            jax.ShapeDtypeStruct((_B, 1), jnp.int32),
        ],
        scratch_shapes=[
            pltpu.VMEM((_BB, _K), jnp.float32),
            pltpu.VMEM((_BB, _K), jnp.float32),
        ],
        compiler_params=pltpu.CompilerParams(
            dimension_semantics=("parallel",),
        ),
    )(cdirn, rhs4, tpf)
    return out_f, idx[:, 0], cnt[:, 0]
